# Initial kernel scaffold; baseline (speedup 1.0000x reference)
import jax
import jax.numpy as jnp
from jax.experimental import pallas as pl


def kernel(inputs, rel_emb_v_weight):
    raise NotImplementedError("write your pallas kernel here")


# speedup vs baseline: 4.9343x; 4.9343x over previous
import functools

import jax
import jax.numpy as jnp
from jax import lax
from jax.experimental import pallas as pl
from jax.experimental.pallas import tpu as pltpu
from jax.experimental.pallas import tpu_sc as plsc

_D = 64
_REP = 8
_VOCAB = 130
_SLOT = 136
_PAD = 65
_SCALE = 8.0

_NC = 2
_NS = 16
_NW = _NC * _NS
_B = 512 * 512
_BPW = _B // _NW
_CH = 128
_NCHUNK = _BPW // _CH
_W = 2 * _D


def _emb_body(tbl_hbm, idx_hbm, out_hbm, tscr_hbm, tbl_v, tbl2_v, idx_v,
              rows_v, sem):
    wid = lax.axis_index("s") * _NC + lax.axis_index("c")

    pltpu.sync_copy(tbl_hbm, tbl_v)

    def prep_row(i, carry):
        keep = (i != _PAD).astype(jnp.float32) * _SCALE
        for k in range(_D // 16):
            x = tbl_v[i, pl.ds(k * 16, 16)] * keep
            tbl2_v[i, pl.ds(k * 16, 16)] = x
            tbl2_v[i, pl.ds(_D + k * 16, 16)] = x
        return carry

    lax.fori_loop(0, _VOCAB, prep_row, 0)
    pltpu.sync_copy(tbl2_v, tscr_hbm.at[pl.ds(wid * _SLOT, _SLOT)])

    pltpu.sync_copy(idx_hbm.at[wid], idx_v)

    def chunk(g, carry):
        off = wid * _SLOT
        for k in range(_CH // 16):
            idx_v[g, pl.ds(k * 16, 16)] = idx_v[g, pl.ds(k * 16, 16)] + off
        pltpu.async_copy(tscr_hbm.at[idx_v.at[g]], rows_v, sem).wait()
        base = wid * _BPW + g * _CH
        for r in range(_REP // 2):
            pltpu.sync_copy(
                rows_v, out_hbm.at[pl.ds(base, _CH), pl.ds(r * _W, _W)])
        return carry

    lax.fori_loop(0, _NCHUNK, chunk, 0)


_emb_kernel = functools.partial(
    pl.kernel,
    out_type=(
        jax.ShapeDtypeStruct((_B, _REP * _D), jnp.float32),
        jax.ShapeDtypeStruct((_NW * _SLOT, _W), jnp.float32),
    ),
    mesh=plsc.VectorSubcoreMesh(core_axis_name="c", subcore_axis_name="s"),
    scratch_types=[
        pltpu.VMEM((_VOCAB, _D), jnp.float32),
        pltpu.VMEM((_SLOT, _W), jnp.float32),
        pltpu.VMEM((_NCHUNK, _CH), jnp.int32),
        pltpu.VMEM((_CH, _W), jnp.float32),
        pltpu.SemaphoreType.DMA,
    ],
)(_emb_body)


def kernel(inputs, rel_emb_v_weight):
    idx = inputs.reshape(_NW, _NCHUNK, _CH).astype(jnp.int32)
    out, _ = _emb_kernel(rel_emb_v_weight.astype(jnp.float32), idx)
    return out.reshape(1, 512, 512, _REP * _D)


# speedup vs baseline: 5.3305x; 1.0803x over previous
import functools

import jax
import jax.numpy as jnp
from jax import lax
from jax.experimental import pallas as pl
from jax.experimental.pallas import tpu as pltpu
from jax.experimental.pallas import tpu_sc as plsc

_D = 64
_REP = 8
_VOCAB = 130
_SLOT = 136
_PAD = 65
_SCALE = 8.0

_NC = 2
_NS = 16
_NW = _NC * _NS
_B = 512 * 512
_BPW = _B // _NW
_CH = 128
_NCHUNK = _BPW // _CH
_W = 2 * _D


_NBUF = 4
_NWR = _REP // 2


def _emb_body(tbl_hbm, idx_hbm, out_hbm, tscr_hbm, tbl_v, tbl2_v, idx_v,
              *bufs_and_sems):
    rows = bufs_and_sems[:_NBUF]
    sem_g = bufs_and_sems[_NBUF:2 * _NBUF]
    sem_w = bufs_and_sems[2 * _NBUF:]
    wid = lax.axis_index("s") * _NC + lax.axis_index("c")

    pltpu.sync_copy(tbl_hbm, tbl_v)

    def prep_row(i, carry):
        keep = (i != _PAD).astype(jnp.float32) * _SCALE
        for k in range(_D // 16):
            x = tbl_v[i, pl.ds(k * 16, 16)] * keep
            tbl2_v[i, pl.ds(k * 16, 16)] = x
            tbl2_v[i, pl.ds(_D + k * 16, 16)] = x
        return carry

    lax.fori_loop(0, _VOCAB, prep_row, 0)
    pltpu.sync_copy(tbl2_v, tscr_hbm.at[pl.ds(wid * _SLOT, _SLOT)])

    pltpu.sync_copy(idx_hbm.at[wid], idx_v)
    off = wid * _SLOT

    def off_row(t, carry):
        for k in range(_CH // 16):
            idx_v[t, pl.ds(k * 16, 16)] = idx_v[t, pl.ds(k * 16, 16)] + off
        return carry

    lax.fori_loop(0, _NCHUNK, off_row, 0)

    def chunk_group(g, carry):
        handles = []
        for b in range(_NBUF):
            t = g * _NBUF + b

            @pl.when(g > 0)
            def _drain(b=b):
                for _ in range(_NWR):
                    pltpu.make_async_copy(
                        rows[b],
                        out_hbm.at[pl.ds(0, _CH), pl.ds(0, _W)],
                        sem_w[b]).wait()

            handles.append(
                pltpu.async_copy(tscr_hbm.at[idx_v.at[t]], rows[b], sem_g[b]))
        for b in range(_NBUF):
            t = g * _NBUF + b
            base = wid * _BPW + t * _CH
            handles[b].wait()
            for r in range(_NWR):
                pltpu.async_copy(
                    rows[b], out_hbm.at[pl.ds(base, _CH), pl.ds(r * _W, _W)],
                    sem_w[b])
        return carry

    lax.fori_loop(0, _NCHUNK // _NBUF, chunk_group, 0)
    for b in range(_NBUF):
        for _ in range(_NWR):
            pltpu.make_async_copy(
                rows[b], out_hbm.at[pl.ds(0, _CH), pl.ds(0, _W)],
                sem_w[b]).wait()


_emb_kernel = functools.partial(
    pl.kernel,
    out_type=(
        jax.ShapeDtypeStruct((_B, _REP * _D), jnp.float32),
        jax.ShapeDtypeStruct((_NW * _SLOT, _W), jnp.float32),
    ),
    mesh=plsc.VectorSubcoreMesh(core_axis_name="c", subcore_axis_name="s"),
    scratch_types=(
        [
            pltpu.VMEM((_VOCAB, _D), jnp.float32),
            pltpu.VMEM((_SLOT, _W), jnp.float32),
            pltpu.VMEM((_NCHUNK, _CH), jnp.int32),
        ]
        + [pltpu.VMEM((_CH, _W), jnp.float32) for _ in range(_NBUF)]
        + [pltpu.SemaphoreType.DMA for _ in range(2 * _NBUF)]
    ),
)(_emb_body)


def kernel(inputs, rel_emb_v_weight):
    idx = inputs.reshape(_NW, _NCHUNK, _CH).astype(jnp.int32)
    out, _ = _emb_kernel(rel_emb_v_weight.astype(jnp.float32), idx)
    return out.reshape(1, 512, 512, _REP * _D)


# speedup vs baseline: 7.7865x; 1.4607x over previous
import functools

import jax
import jax.numpy as jnp
from jax import lax
from jax.experimental import pallas as pl
from jax.experimental.pallas import tpu as pltpu
from jax.experimental.pallas import tpu_sc as plsc

_D = 64
_REP = 8
_VOCAB = 130
_SLOT = 136
_PAD = 65
_SCALE = 8.0

_NC = 2
_NS = 16
_NW = _NC * _NS
_B = 512 * 512
_BPW = _B // _NW
_CH = 128
_NCHUNK = _BPW // _CH
_W = 2 * _D


_NBUF = 4
_NWR = _REP // 2


def _emb_body(tbl_hbm, idx_hbm, out_hbm, tbl_v, tbl2_v, idx_v, tscr_spm,
              *bufs_and_sems):
    rows = bufs_and_sems[:_NBUF]
    sem_g = bufs_and_sems[_NBUF:2 * _NBUF]
    sem_w = bufs_and_sems[2 * _NBUF:]
    wid = lax.axis_index("s") * _NC + lax.axis_index("c")

    pltpu.sync_copy(tbl_hbm, tbl_v)

    def prep_row(i, carry):
        keep = (i != _PAD).astype(jnp.float32) * _SCALE
        for k in range(_D // 16):
            x = tbl_v[i, pl.ds(k * 16, 16)] * keep
            tbl2_v[i, pl.ds(k * 16, 16)] = x
            tbl2_v[i, pl.ds(_D + k * 16, 16)] = x
        return carry

    lax.fori_loop(0, _VOCAB, prep_row, 0)
    sid = lax.axis_index("s")
    pltpu.sync_copy(tbl2_v, tscr_spm.at[pl.ds(sid * _SLOT, _SLOT)])

    pltpu.sync_copy(idx_hbm.at[wid], idx_v)
    off = sid * _SLOT

    def off_row(t, carry):
        for k in range(_CH // 16):
            idx_v[t, pl.ds(k * 16, 16)] = idx_v[t, pl.ds(k * 16, 16)] + off
        return carry

    lax.fori_loop(0, _NCHUNK, off_row, 0)

    def chunk_group(g, carry):
        handles = []
        for b in range(_NBUF):
            t = g * _NBUF + b

            @pl.when(g > 0)
            def _drain(b=b):
                for _ in range(_NWR):
                    pltpu.make_async_copy(
                        rows[b],
                        out_hbm.at[pl.ds(0, _CH), pl.ds(0, _W)],
                        sem_w[b]).wait()

            handles.append(
                pltpu.async_copy(tscr_spm.at[idx_v.at[t]], rows[b], sem_g[b]))
        for b in range(_NBUF):
            t = g * _NBUF + b
            base = wid * _BPW + t * _CH
            handles[b].wait()
            for r in range(_NWR):
                pltpu.async_copy(
                    rows[b], out_hbm.at[pl.ds(base, _CH), pl.ds(r * _W, _W)],
                    sem_w[b])
        return carry

    lax.fori_loop(0, _NCHUNK // _NBUF, chunk_group, 0)
    for b in range(_NBUF):
        for _ in range(_NWR):
            pltpu.make_async_copy(
                rows[b], out_hbm.at[pl.ds(0, _CH), pl.ds(0, _W)],
                sem_w[b]).wait()


_emb_kernel = functools.partial(
    pl.kernel,
    out_type=jax.ShapeDtypeStruct((_B, _REP * _D), jnp.float32),
    mesh=plsc.VectorSubcoreMesh(core_axis_name="c", subcore_axis_name="s"),
    scratch_types=(
        [
            pltpu.VMEM((_VOCAB, _D), jnp.float32),
            pltpu.VMEM((_SLOT, _W), jnp.float32),
            pltpu.VMEM((_NCHUNK, _CH), jnp.int32),
            pltpu.VMEM_SHARED((_NS * _SLOT, _W), jnp.float32),
        ]
        + [pltpu.VMEM((_CH, _W), jnp.float32) for _ in range(_NBUF)]
        + [pltpu.SemaphoreType.DMA for _ in range(2 * _NBUF)]
    ),
)(_emb_body)


def kernel(inputs, rel_emb_v_weight):
    idx = inputs.reshape(_NW, _NCHUNK, _CH).astype(jnp.int32)
    out = _emb_kernel(rel_emb_v_weight.astype(jnp.float32), idx)
    return out.reshape(1, 512, 512, _REP * _D)


# speedup vs baseline: 7.9228x; 1.0175x over previous
import functools

import jax
import jax.numpy as jnp
from jax import lax
from jax.experimental import pallas as pl
from jax.experimental.pallas import tpu as pltpu
from jax.experimental.pallas import tpu_sc as plsc

_D = 64
_REP = 8
_VOCAB = 130
_SLOT = 136
_PAD = 65
_SCALE = 8.0

_NC = 2
_NS = 16
_NW = _NC * _NS
_B = 512 * 512
_BPW = _B // _NW
_CH = 128
_NCHUNK = _BPW // _CH
_W = 2 * _D

_NBUF = 4
_NWR = _REP * _D // _W


def _emb_body(tbl_hbm, idx_hbm, out_hbm, tbl_v, tbl2_v, idx_v, tscr_spm,
              *bufs_and_sems):
    rows = bufs_and_sems[:_NBUF]
    sem_g = bufs_and_sems[_NBUF:2 * _NBUF]
    sem_w = bufs_and_sems[2 * _NBUF:]
    sid = lax.axis_index("s")
    wid = sid * _NC + lax.axis_index("c")

    @pl.when(sid == 0)
    def _prep():
        pltpu.sync_copy(tbl_hbm, tbl_v)

        def prep_row(i, carry):
            keep = (i != _PAD).astype(jnp.float32) * _SCALE
            for k in range(_D // 16):
                x = tbl_v[i, pl.ds(k * 16, 16)] * keep
                for rep in range(_W // _D):
                    tbl2_v[i, pl.ds(rep * _D + k * 16, 16)] = x
            return carry

        lax.fori_loop(0, _VOCAB, prep_row, 0)
        pltpu.sync_copy(tbl2_v, tscr_spm)

    pltpu.sync_copy(idx_hbm.at[wid], idx_v)
    plsc.subcore_barrier()

    def chunk_group(g, carry):
        handles = []
        for b in range(_NBUF):
            t = g * _NBUF + b

            @pl.when(g > 0)
            def _drain(b=b):
                for _ in range(_NWR):
                    pltpu.make_async_copy(
                        rows[b],
                        out_hbm.at[pl.ds(0, _CH), pl.ds(0, _W)],
                        sem_w[b]).wait()

            handles.append(
                pltpu.async_copy(tscr_spm.at[idx_v.at[t]], rows[b], sem_g[b]))
        for b in range(_NBUF):
            t = g * _NBUF + b
            base = wid * _BPW + t * _CH
            handles[b].wait()
            for r in range(_NWR):
                pltpu.async_copy(
                    rows[b], out_hbm.at[pl.ds(base, _CH), pl.ds(r * _W, _W)],
                    sem_w[b])
        return carry

    lax.fori_loop(0, _NCHUNK // _NBUF, chunk_group, 0)
    for b in range(_NBUF):
        for _ in range(_NWR):
            pltpu.make_async_copy(
                rows[b], out_hbm.at[pl.ds(0, _CH), pl.ds(0, _W)],
                sem_w[b]).wait()


_emb_kernel = functools.partial(
    pl.kernel,
    out_type=jax.ShapeDtypeStruct((_B, _REP * _D), jnp.float32),
    mesh=plsc.VectorSubcoreMesh(core_axis_name="c", subcore_axis_name="s"),
    scratch_types=(
        [
            pltpu.VMEM((_VOCAB, _D), jnp.float32),
            pltpu.VMEM((_SLOT, _W), jnp.float32),
            pltpu.VMEM((_NCHUNK, _CH), jnp.int32),
            pltpu.VMEM_SHARED((_SLOT, _W), jnp.float32),
        ]
        + [pltpu.VMEM((_CH, _W), jnp.float32) for _ in range(_NBUF)]
        + [pltpu.SemaphoreType.DMA for _ in range(2 * _NBUF)]
    ),
)(_emb_body)


def kernel(inputs, rel_emb_v_weight):
    idx = inputs.reshape(_NW, _NCHUNK, _CH).astype(jnp.int32)
    out = _emb_kernel(rel_emb_v_weight.astype(jnp.float32), idx)
    return out.reshape(1, 512, 512, _REP * _D)
